# baseline (speedup 1.0000x reference)
import functools

import jax
import jax.numpy as jnp
from jax import lax
from jax.experimental import pallas as pl
from jax.experimental.pallas import tpu as pltpu
from jax.experimental.pallas import tpu_sc as plsc

NUM_EMB = 1_000_000
DIM = 64
PADW = 128
BATCH = 16384
HIST = 50
TOTAL = BATCH * HIST

NUM_CORES = 2
NUM_SUBCORES = 16
NW = NUM_CORES * NUM_SUBCORES

UNIT = 128
NUNIT = TOTAL // UNIT
UNITS_PER_W = NUNIT // NW
BLOCKS_PER_H = BATCH // UNIT

_mesh = plsc.VectorSubcoreMesh(core_axis_name="c", subcore_axis_name="s")


@functools.partial(
    pl.kernel,
    mesh=_mesh,
    out_type=jax.ShapeDtypeStruct((HIST, DIM, BATCH), jnp.float32),
    scratch_types=[
        pltpu.VMEM((UNITS_PER_W, UNIT), jnp.int32),
        pltpu.VMEM((UNIT, PADW), jnp.float32),
        pltpu.VMEM((UNIT, PADW), jnp.float32),
        pltpu.VMEM((DIM, UNIT), jnp.float32),
        pltpu.VMEM((DIM, UNIT), jnp.float32),
        pltpu.SemaphoreType.DMA,
        pltpu.SemaphoreType.DMA,
        pltpu.SemaphoreType.DMA,
        pltpu.SemaphoreType.DMA,
    ],
    compiler_params=pltpu.CompilerParams(
        use_tc_tiling_on_sc=True, needs_layout_passes=False
    ),
)
def _embed_sc(idx_hbm, table_hbm, out_hbm,
              idx_v, p0, p1, ob0, ob1,
              g0, g1, o0, o1):
    prows = (p0, p1)
    obuf = (ob0, ob1)
    gsem = (g0, g1)
    osem = (o0, o1)

    wid = lax.axis_index("s") * NUM_CORES + lax.axis_index("c")
    base_u = wid * UNITS_PER_W

    pltpu.sync_copy(idx_hbm.at[pl.ds(base_u, UNITS_PER_W)], idx_v)

    lane = lax.iota(jnp.int32, 16)
    row_idx = [lane + bg * 16 for bg in range(8)]

    def issue_gather(lu, slot):
        pltpu.async_copy(table_hbm.at[idx_v.at[lu]], prows[slot], gsem[slot])

    def wait_gather(slot):
        pltpu.make_async_copy(
            table_hbm.at[pl.ds(0, UNIT)], prows[slot], gsem[slot]
        ).wait()

    def transpose_unit(slot):
        prow = prows[slot]
        ob = obuf[slot]

        def dbody(d, carry):
            col = jnp.full((16,), d, dtype=jnp.int32)
            for bg in range(8):
                v = plsc.load_gather(prow, [row_idx[bg], col])
                ob[d, pl.ds(bg * 16, 16)] = v
            return carry

        lax.fori_loop(0, DIM, dbody, 0)

    def issue_store(lu, slot):
        gu = base_u + lu
        h = gu // BLOCKS_PER_H
        c = gu % BLOCKS_PER_H
        pltpu.async_copy(
            obuf[slot],
            out_hbm.at[h, pl.ds(0, DIM), pl.ds(c * UNIT, UNIT)],
            osem[slot],
        )

    def wait_store(slot):
        pltpu.make_async_copy(
            obuf[slot], out_hbm.at[0, pl.ds(0, DIM), pl.ds(0, UNIT)],
            osem[slot],
        ).wait()

    issue_gather(0, 0)
    issue_gather(1, 1)
    wait_gather(0); transpose_unit(0); issue_store(0, 0); issue_gather(2, 0)
    wait_gather(1); transpose_unit(1); issue_store(1, 1); issue_gather(3, 1)

    def body(t, carry):
        lu0 = t * 2
        for j in range(2):
            lu = lu0 + j
            wait_store(j)
            wait_gather(j)
            transpose_unit(j)
            issue_store(lu, j)
            issue_gather(lu + 2, j)
        return carry

    lax.fori_loop(1, UNITS_PER_W // 2 - 1, body, 0)

    cL = UNITS_PER_W - 2
    wait_store(0); wait_gather(0); transpose_unit(0); issue_store(cL + 0, 0)
    wait_store(1); wait_gather(1); transpose_unit(1); issue_store(cL + 1, 1)
    wait_store(0)
    wait_store(1)


def kernel(token_ids, weight):
    wpad = jnp.pad(weight, ((0, 0), (0, PADW - DIM)))
    idx = token_ids.T.reshape(NUNIT, UNIT)
    out = _embed_sc(idx, wpad)
    return out.transpose(2, 0, 1)


# speedup vs baseline: 1.4190x; 1.4190x over previous
import functools

import jax
import jax.numpy as jnp
from jax import lax
from jax.experimental import pallas as pl
from jax.experimental.pallas import tpu as pltpu
from jax.experimental.pallas import tpu_sc as plsc

NUM_EMB = 1_000_000
DIM = 64
PADW = 128
BATCH = 16384
HIST = 50
TOTAL = BATCH * HIST

NUM_CORES = 2
NUM_SUBCORES = 16
NW = NUM_CORES * NUM_SUBCORES
ROWS_PER_W = TOTAL // NW

IDX_MINOR = 128
CHUNK = 128
NCHUNK = ROWS_PER_W // CHUNK
NBUF = 4
NOUTER = NCHUNK // NBUF
IDX_ROWS = ROWS_PER_W // IDX_MINOR
CHUNKS_PER_H = BATCH // CHUNK

_mesh = plsc.VectorSubcoreMesh(core_axis_name="c", subcore_axis_name="s")


@functools.partial(
    pl.kernel,
    mesh=_mesh,
    out_type=jax.ShapeDtypeStruct((HIST, BATCH, PADW), jnp.float32),
    scratch_types=[
        pltpu.VMEM((IDX_ROWS, IDX_MINOR), jnp.int32),
        pltpu.VMEM((CHUNK, PADW), jnp.float32),
        pltpu.VMEM((CHUNK, PADW), jnp.float32),
        pltpu.VMEM((CHUNK, PADW), jnp.float32),
        pltpu.VMEM((CHUNK, PADW), jnp.float32),
        pltpu.SemaphoreType.DMA,
        pltpu.SemaphoreType.DMA,
        pltpu.SemaphoreType.DMA,
        pltpu.SemaphoreType.DMA,
        pltpu.SemaphoreType.DMA,
        pltpu.SemaphoreType.DMA,
        pltpu.SemaphoreType.DMA,
        pltpu.SemaphoreType.DMA,
    ],
    compiler_params=pltpu.CompilerParams(
        use_tc_tiling_on_sc=True, needs_layout_passes=False
    ),
)
def _embed_sc(idx_hbm, table_hbm, out_hbm,
              idx_v, r0, r1, r2, r3,
              g0, g1, g2, g3, o0, o1, o2, o3):
    rows = (r0, r1, r2, r3)
    gsem = (g0, g1, g2, g3)
    osem = (o0, o1, o2, o3)

    wid = lax.axis_index("s") * NUM_CORES + lax.axis_index("c")
    base_chunk = wid * NCHUNK
    base_blk = wid * IDX_ROWS

    pltpu.sync_copy(idx_hbm.at[pl.ds(base_blk, IDX_ROWS)], idx_v)

    def issue_gather(c, slot):
        pltpu.async_copy(
            table_hbm.at[idx_v.at[c]],
            rows[slot],
            gsem[slot],
        )

    def wait_gather(slot):
        pltpu.make_async_copy(
            table_hbm.at[pl.ds(0, CHUNK)], rows[slot], gsem[slot]
        ).wait()

    def issue_store(c, slot):
        gc = base_chunk + c
        h = gc // CHUNKS_PER_H
        b0 = (gc % CHUNKS_PER_H) * CHUNK
        pltpu.async_copy(
            rows[slot],
            out_hbm.at[h, pl.ds(b0, CHUNK)],
            osem[slot],
        )

    def wait_store(slot):
        pltpu.make_async_copy(
            rows[slot], out_hbm.at[0, pl.ds(0, CHUNK)], osem[slot]
        ).wait()

    issue_gather(0, 0)
    issue_gather(1, 1)
    wait_gather(0); issue_store(0, 0); issue_gather(2, 2)
    wait_gather(1); issue_store(1, 1); issue_gather(3, 3)
    wait_gather(2); issue_store(2, 2); wait_store(0); issue_gather(4, 0)
    wait_gather(3); issue_store(3, 3); wait_store(1); issue_gather(5, 1)

    def body(t, carry):
        c0 = t * NBUF
        for j in range(NBUF):
            c = c0 + j
            wait_gather(j)
            issue_store(c, j)
            wait_store((j + 2) % NBUF)
            issue_gather(c + 2, (j + 2) % NBUF)
        return carry

    lax.fori_loop(1, NOUTER - 1, body, 0)

    cL = (NOUTER - 1) * NBUF
    wait_gather(0); issue_store(cL + 0, 0); wait_store(2); issue_gather(cL + 2, 2)
    wait_gather(1); issue_store(cL + 1, 1); wait_store(3); issue_gather(cL + 3, 3)
    wait_gather(2); issue_store(cL + 2, 2); wait_store(0)
    wait_gather(3); issue_store(cL + 3, 3); wait_store(1)
    wait_store(2)
    wait_store(3)


def kernel(token_ids, weight):
    wpad = jnp.pad(weight, ((0, 0), (0, PADW - DIM)))
    idx = token_ids.T.reshape(TOTAL // IDX_MINOR, IDX_MINOR)
    out = _embed_sc(idx, wpad)
    return out[:, :, :DIM].transpose(1, 0, 2)


# speedup vs baseline: 1.5725x; 1.1081x over previous
import functools

import jax
import jax.numpy as jnp
from jax import lax
from jax.experimental import pallas as pl
from jax.experimental.pallas import tpu as pltpu
from jax.experimental.pallas import tpu_sc as plsc

NUM_EMB = 1_000_000
DIM = 64
PADW = 128
BATCH = 16384
HIST = 50
TOTAL = BATCH * HIST

NUM_CORES = 2
NUM_SUBCORES = 16
NW = NUM_CORES * NUM_SUBCORES

UNIT = 128
NUNIT = TOTAL // UNIT
UNITS_PER_W = NUNIT // NW
BLOCKS_PER_H = BATCH // UNIT

_mesh = plsc.VectorSubcoreMesh(core_axis_name="c", subcore_axis_name="s")


@functools.partial(
    pl.kernel,
    mesh=_mesh,
    out_type=jax.ShapeDtypeStruct((HIST, DIM, BATCH), jnp.float32),
    scratch_types=[
        pltpu.VMEM((UNITS_PER_W, UNIT), jnp.int32),
        pltpu.VMEM((UNIT, PADW), jnp.float32),
        pltpu.VMEM((UNIT, PADW), jnp.float32),
        pltpu.VMEM((DIM, UNIT), jnp.float32),
        pltpu.VMEM((DIM, UNIT), jnp.float32),
        pltpu.SemaphoreType.DMA,
        pltpu.SemaphoreType.DMA,
        pltpu.SemaphoreType.DMA,
        pltpu.SemaphoreType.DMA,
    ],
    compiler_params=pltpu.CompilerParams(
        use_tc_tiling_on_sc=True, needs_layout_passes=False
    ),
)
def _embed_sc(idx_hbm, table_hbm, out_hbm,
              idx_v, p0, p1, ob0, ob1,
              g0, g1, o0, o1):
    prows = (p0, p1)
    obuf = (ob0, ob1)
    gsem = (g0, g1)
    osem = (o0, o1)

    wid = lax.axis_index("s") * NUM_CORES + lax.axis_index("c")
    base_u = wid * UNITS_PER_W

    pltpu.sync_copy(idx_hbm.at[pl.ds(base_u, UNITS_PER_W)], idx_v)

    lane = lax.iota(jnp.int32, 16)
    rowv = [lane + bg * 16 for bg in range(8)]
    diag = [jnp.bitwise_and(lane + k, 15) for k in range(16)]

    def issue_gather(lu, slot):
        pltpu.async_copy(table_hbm.at[idx_v.at[lu]], prows[slot], gsem[slot])

    def wait_gather(slot):
        pltpu.make_async_copy(
            table_hbm.at[pl.ds(0, UNIT)], prows[slot], gsem[slot]
        ).wait()

    def transpose_unit(slot):
        prow = prows[slot]
        ob = obuf[slot]

        def dgbody(dg, carry):
            dgo = dg * 16
            for bg in range(8):
                for k in range(16):
                    colv = diag[k] + dgo
                    v = plsc.load_gather(prow, [rowv[bg], colv])
                    plsc.store_scatter(ob, [colv, rowv[bg]], v)
            return carry

        lax.fori_loop(0, DIM // 16, dgbody, 0)

    def issue_store(lu, slot):
        gu = base_u + lu
        h = gu // BLOCKS_PER_H
        c = gu % BLOCKS_PER_H
        pltpu.async_copy(
            obuf[slot],
            out_hbm.at[h, pl.ds(0, DIM), pl.ds(c * UNIT, UNIT)],
            osem[slot],
        )

    def wait_store(slot):
        pltpu.make_async_copy(
            obuf[slot], out_hbm.at[0, pl.ds(0, DIM), pl.ds(0, UNIT)],
            osem[slot],
        ).wait()

    issue_gather(0, 0)
    issue_gather(1, 1)
    wait_gather(0); transpose_unit(0); issue_store(0, 0); issue_gather(2, 0)
    wait_gather(1); transpose_unit(1); issue_store(1, 1); issue_gather(3, 1)

    def body(t, carry):
        lu0 = t * 2
        for j in range(2):
            lu = lu0 + j
            wait_store(j)
            wait_gather(j)
            transpose_unit(j)
            issue_store(lu, j)
            issue_gather(lu + 2, j)
        return carry

    lax.fori_loop(1, UNITS_PER_W // 2 - 1, body, 0)

    cL = UNITS_PER_W - 2
    wait_store(0); wait_gather(0); transpose_unit(0); issue_store(cL + 0, 0)
    wait_store(1); wait_gather(1); transpose_unit(1); issue_store(cL + 1, 1)
    wait_store(0)
    wait_store(1)


def kernel(token_ids, weight):
    wpad = jnp.pad(weight, ((0, 0), (0, PADW - DIM)))
    idx = token_ids.T.reshape(NUNIT, UNIT)
    out = _embed_sc(idx, wpad)
    return out.transpose(2, 0, 1)


# speedup vs baseline: 1.6503x; 1.0495x over previous
import functools

import jax
import jax.numpy as jnp
from jax import lax
from jax.experimental import pallas as pl
from jax.experimental.pallas import tpu as pltpu
from jax.experimental.pallas import tpu_sc as plsc

NUM_EMB = 1_000_000
DIM = 64
PADW = 128
BATCH = 16384
HIST = 50
TOTAL = BATCH * HIST

NUM_CORES = 2
NUM_SUBCORES = 16
NW = NUM_CORES * NUM_SUBCORES

UNIT = 128
NUNIT = TOTAL // UNIT
UNITS_PER_W = NUNIT // NW
BLOCKS_PER_H = BATCH // UNIT

_mesh = plsc.VectorSubcoreMesh(core_axis_name="c", subcore_axis_name="s")


@functools.partial(
    pl.kernel,
    mesh=_mesh,
    out_type=jax.ShapeDtypeStruct((HIST, DIM, BATCH), jnp.float32),
    scratch_types=[
        pltpu.VMEM((UNITS_PER_W, UNIT), jnp.int32),
        pltpu.VMEM((UNIT, PADW), jnp.float32),
        pltpu.VMEM((UNIT, PADW), jnp.float32),
        pltpu.VMEM((DIM, UNIT), jnp.float32),
        pltpu.VMEM((DIM, UNIT), jnp.float32),
        pltpu.SemaphoreType.DMA,
        pltpu.SemaphoreType.DMA,
        pltpu.SemaphoreType.DMA,
        pltpu.SemaphoreType.DMA,
    ],
    compiler_params=pltpu.CompilerParams(
        use_tc_tiling_on_sc=True, needs_layout_passes=False
    ),
)
def _embed_sc(idx_hbm, table_hbm, out_hbm,
              idx_v, p0, p1, ob0, ob1,
              g0, g1, o0, o1):
    prows = (p0, p1)
    obuf = (ob0, ob1)
    gsem = (g0, g1)
    osem = (o0, o1)

    wid = lax.axis_index("s") * NUM_CORES + lax.axis_index("c")
    base_u = wid * UNITS_PER_W

    pltpu.sync_copy(idx_hbm.at[pl.ds(base_u, UNITS_PER_W)], idx_v)

    lane = lax.iota(jnp.int32, 16)
    rowv = [lane + bg * 16 for bg in range(8)]
    diag = [jnp.bitwise_and(lane + k, 15) for k in range(16)]

    def issue_gather(lu, slot):
        pltpu.async_copy(table_hbm.at[idx_v.at[lu]], prows[slot], gsem[slot])

    def wait_gather(slot):
        pltpu.make_async_copy(
            table_hbm.at[pl.ds(0, UNIT)], prows[slot], gsem[slot]
        ).wait()

    def transpose_unit(slot):
        prow = prows[slot]
        ob = obuf[slot]

        def dgbody(dg, carry):
            dgo = dg * 16
            for k in range(16):
                colv = diag[k] + dgo
                for bg in range(8):
                    v = plsc.load_gather(prow, [rowv[bg], colv])
                    plsc.store_scatter(ob, [colv, rowv[bg]], v)
            return carry

        lax.fori_loop(0, DIM // 16, dgbody, 0)

    def issue_store(lu, slot):
        gu = base_u + lu
        h = gu // BLOCKS_PER_H
        c = gu % BLOCKS_PER_H
        pltpu.async_copy(
            obuf[slot],
            out_hbm.at[h, pl.ds(0, DIM), pl.ds(c * UNIT, UNIT)],
            osem[slot],
        )

    def wait_store(slot):
        pltpu.make_async_copy(
            obuf[slot], out_hbm.at[0, pl.ds(0, DIM), pl.ds(0, UNIT)],
            osem[slot],
        ).wait()

    issue_gather(0, 0)
    issue_gather(1, 1)
    wait_gather(0); transpose_unit(0); issue_store(0, 0); issue_gather(2, 0)
    wait_gather(1); transpose_unit(1); issue_store(1, 1); issue_gather(3, 1)

    def body(t, carry):
        lu0 = t * 2
        for j in range(2):
            lu = lu0 + j
            wait_store(j)
            wait_gather(j)
            transpose_unit(j)
            issue_store(lu, j)
            issue_gather(lu + 2, j)
        return carry

    lax.fori_loop(1, UNITS_PER_W // 2 - 1, body, 0)

    cL = UNITS_PER_W - 2
    wait_store(0); wait_gather(0); transpose_unit(0); issue_store(cL + 0, 0)
    wait_store(1); wait_gather(1); transpose_unit(1); issue_store(cL + 1, 1)
    wait_store(0)
    wait_store(1)


def kernel(token_ids, weight):
    wpad = jnp.pad(weight, ((0, 0), (0, PADW - DIM)))
    idx = token_ids.T.reshape(NUNIT, UNIT)
    out = _embed_sc(idx, wpad)
    return out.transpose(2, 0, 1)


# speedup vs baseline: 2.1869x; 1.3251x over previous
import functools

import jax
import jax.numpy as jnp
from jax import lax
from jax.experimental import pallas as pl
from jax.experimental.pallas import tpu as pltpu
from jax.experimental.pallas import tpu_sc as plsc

NUM_EMB = 1_000_000
DIM = 64
PADW = 128
BATCH = 16384
HIST = 50
TOTAL = BATCH * HIST

NUM_CORES = 2
NUM_SUBCORES = 16
NW = NUM_CORES * NUM_SUBCORES

UNIT = 128
NUNIT = TOTAL // UNIT
UNITS_PER_W = NUNIT // NW
BLOCKS_PER_H = BATCH // UNIT

_mesh = plsc.VectorSubcoreMesh(core_axis_name="c", subcore_axis_name="s")


@functools.partial(
    pl.kernel,
    mesh=_mesh,
    out_type=jax.ShapeDtypeStruct((HIST, DIM, BATCH), jnp.float32),
    scratch_types=[
        pltpu.VMEM((UNITS_PER_W, UNIT), jnp.int32),
        pltpu.VMEM((UNIT, PADW), jnp.float32),
        pltpu.VMEM((UNIT, PADW), jnp.float32),
        pltpu.VMEM((DIM, UNIT), jnp.float32),
        pltpu.VMEM((DIM, UNIT), jnp.float32),
        pltpu.SemaphoreType.DMA,
        pltpu.SemaphoreType.DMA,
        pltpu.SemaphoreType.DMA,
        pltpu.SemaphoreType.DMA,
    ],
    compiler_params=pltpu.CompilerParams(
        use_tc_tiling_on_sc=True, needs_layout_passes=False
    ),
)
def _embed_sc(idx_hbm, table_hbm, out_hbm,
              idx_v, p0, p1, ob0, ob1,
              g0, g1, o0, o1):
    prows = (p0, p1)
    obuf = (ob0, ob1)
    gsem = (g0, g1)
    osem = (o0, o1)

    wid = lax.axis_index("s") * NUM_CORES + lax.axis_index("c")
    base_u = wid * UNITS_PER_W

    pltpu.sync_copy(idx_hbm.at[pl.ds(base_u, UNITS_PER_W)], idx_v)

    lane = lax.iota(jnp.int32, 16)
    rowv = [lane + bg * 16 for bg in range(8)]
    diag = [jnp.bitwise_and(lane + k, 15) for k in range(16)]

    def issue_gather(lu, slot):
        pltpu.async_copy(table_hbm.at[idx_v.at[lu]], prows[slot], gsem[slot])

    def wait_gather(slot):
        pltpu.make_async_copy(
            table_hbm.at[pl.ds(0, UNIT)], prows[slot], gsem[slot]
        ).wait()

    def transpose_unit(slot):
        prow = prows[slot]
        ob = obuf[slot]

        def dgbody(dg, carry):
            dgo = dg * 16
            for k in range(16):
                colv = diag[k] + dgo
                vs = [plsc.load_gather(prow, [rowv[bg], colv])
                      for bg in range(8)]
                for bg in range(8):
                    plsc.store_scatter(ob, [colv, rowv[bg]], vs[bg])
            return carry

        lax.fori_loop(0, DIM // 16, dgbody, 0)

    def issue_store(lu, slot):
        gu = base_u + lu
        h = gu // BLOCKS_PER_H
        c = gu % BLOCKS_PER_H
        pltpu.async_copy(
            obuf[slot],
            out_hbm.at[h, pl.ds(0, DIM), pl.ds(c * UNIT, UNIT)],
            osem[slot],
        )

    def wait_store(slot):
        pltpu.make_async_copy(
            obuf[slot], out_hbm.at[0, pl.ds(0, DIM), pl.ds(0, UNIT)],
            osem[slot],
        ).wait()

    issue_gather(0, 0)
    issue_gather(1, 1)
    wait_gather(0); transpose_unit(0); issue_store(0, 0); issue_gather(2, 0)
    wait_gather(1); transpose_unit(1); issue_store(1, 1); issue_gather(3, 1)

    def body(t, carry):
        lu0 = t * 2
        for j in range(2):
            lu = lu0 + j
            wait_store(j)
            wait_gather(j)
            transpose_unit(j)
            issue_store(lu, j)
            issue_gather(lu + 2, j)
        return carry

    lax.fori_loop(1, UNITS_PER_W // 2 - 1, body, 0)

    cL = UNITS_PER_W - 2
    wait_store(0); wait_gather(0); transpose_unit(0); issue_store(cL + 0, 0)
    wait_store(1); wait_gather(1); transpose_unit(1); issue_store(cL + 1, 1)
    wait_store(0)
    wait_store(1)


def kernel(token_ids, weight):
    wpad = jnp.pad(weight, ((0, 0), (0, PADW - DIM)))
    idx = token_ids.T.reshape(NUNIT, UNIT)
    out = _embed_sc(idx, wpad)
    return out.transpose(2, 0, 1)
